# baseline (speedup 1.0000x reference)
import functools

import jax
import jax.numpy as jnp
from jax import lax
from jax.experimental import pallas as pl
from jax.experimental.pallas import tpu as pltpu
from jax.experimental.pallas import tpu_sc as plsc

NC, NS, LANES = 2, 16, 16
NW = NC * NS
D = 128
HA, HB = 96, 104
RES = 16
_SC_PARAMS = pltpu.CompilerParams(use_tc_tiling_on_sc=False,
                                  needs_layout_passes=False)


def _make_convert(V):
    rows_per_w = V // NW
    CH = 125
    nch = rows_per_w // CH
    assert rows_per_w % CH == 0
    mesh = plsc.VectorSubcoreMesh(core_axis_name="c", subcore_axis_name="s")

    @functools.partial(
        pl.kernel,
        out_type=jax.ShapeDtypeStruct((V, D), jnp.bfloat16),
        mesh=mesh,
        compiler_params=_SC_PARAMS,
        scratch_types=[
            pltpu.VMEM((3, CH, D), jnp.float32),
            pltpu.VMEM((3, CH, D), jnp.bfloat16),
            pltpu.SemaphoreType.DMA,
            pltpu.SemaphoreType.DMA,
            pltpu.SemaphoreType.DMA,
            pltpu.SemaphoreType.DMA,
            pltpu.SemaphoreType.DMA,
            pltpu.SemaphoreType.DMA,
        ],
    )
    def convert(table, out, in_v, out_v, si0, si1, si2, so0, so1, so2):
        wid = lax.axis_index("s") * NC + lax.axis_index("c")
        base = wid * rows_per_w
        sis = (si0, si1, si2)
        sos = (so0, so1, so2)

        def in_start(k, b):
            pltpu.make_async_copy(table.at[pl.ds(base + k * CH, CH)],
                                  in_v.at[b], sis[b]).start()

        def in_wait(b):
            pltpu.make_async_copy(table.at[pl.ds(base, CH)],
                                  in_v.at[b], sis[b]).wait()

        def out_start(k, b):
            pltpu.make_async_copy(out_v.at[b],
                                  out.at[pl.ds(base + k * CH, CH)],
                                  sos[b]).start()

        def out_wait(b):
            pltpu.make_async_copy(out_v.at[b],
                                  out.at[pl.ds(base, CH)], sos[b]).wait()

        def convert_chunk(b):
            @plsc.parallel_loop(0, CH, 1, unroll=5)
            def _(r):
                for c in range(D // 32):
                    g0 = in_v[b, r, pl.ds(c * 32, LANES)]
                    g1 = in_v[b, r, pl.ds(c * 32 + LANES, LANES)]
                    out_v[b, r, pl.ds(c * 32, 32)] = plsc.pack(
                        g0, g1, format=plsc.PackFormat.INTERLEAVED)

        in_start(0, 0)
        in_start(1, 1)

        def chunk_body(k, _):
            for q in range(3):
                @pl.when(lax.rem(k, 3) == q)
                def _(q=q):
                    @pl.when(k < nch - 2)
                    def _():
                        in_start(k + 2, (q + 2) % 3)
                    in_wait(q)
                    @pl.when(k >= 3)
                    def _():
                        out_wait(q)
                    convert_chunk(q)
                    out_start(k, q)
            return 0

        lax.fori_loop(0, nch, chunk_body, 0)
        for k in (nch - 3, nch - 2, nch - 1):
            out_wait(k % 3)

    return convert


def _make_seg_sum(S, L, V):
    assert L == HA + HB
    seg_per_w = S // NW
    npairs = seg_per_w // 2
    mesh = plsc.VectorSubcoreMesh(core_axis_name="c", subcore_axis_name="s")

    @functools.partial(
        pl.kernel,
        out_type=jax.ShapeDtypeStruct((S, D), jnp.float32),
        mesh=mesh,
        compiler_params=_SC_PARAMS,
        scratch_types=[
            pltpu.VMEM((seg_per_w * L,), jnp.int32),
            pltpu.VMEM((L, D), jnp.bfloat16),
            pltpu.VMEM((L, D), jnp.bfloat16),
            pltpu.VMEM((L, D), jnp.bfloat16),
            pltpu.VMEM((L, D), jnp.bfloat16),
            pltpu.VMEM((L, D), jnp.bfloat16),
            pltpu.VMEM((RES, D), jnp.float32),
            pltpu.SemaphoreType.DMA,
            pltpu.SemaphoreType.DMA,
            pltpu.SemaphoreType.DMA,
            pltpu.SemaphoreType.DMA,
            pltpu.SemaphoreType.DMA,
        ],
    )
    def seg_sum(table, idx, out, idx_v, rows0, rows1, rows2, rows3, rows4,
                res_v, sem0, sem1, sem2, sem3, sem4):
        wid = lax.axis_index("s") * NC + lax.axis_index("c")
        wseg = wid * seg_per_w

        pltpu.sync_copy(idx.at[pl.ds(wseg * L, seg_per_w * L)], idx_v)

        def g_start(seg, rows, sem):
            off = seg * L
            pltpu.make_async_copy(
                table.at[idx_v.at[pl.ds(off, HA)]],
                rows.at[pl.ds(0, HA)], sem).start()
            pltpu.make_async_copy(
                table.at[idx_v.at[pl.ds(off + HA, HB)]],
                rows.at[pl.ds(HA, HB)], sem).start()

        def g_wait(rows, sem):
            pltpu.make_async_copy(
                table.at[idx_v.at[pl.ds(0, HA)]],
                rows.at[pl.ds(0, HA)], sem).wait()
            pltpu.make_async_copy(
                table.at[idx_v.at[pl.ds(0, HB)]],
                rows.at[pl.ds(HA, HB)], sem).wait()

        zero16 = jnp.zeros((LANES,), jnp.float32)

        def _tree_sum(vs):
            while len(vs) > 1:
                vs = [vs[i] + vs[i + 1] for i in range(0, len(vs) - 1, 2)] \
                    + ([vs[-1]] if len(vs) % 2 else [])
            return vs[0]

        def seg_sum_rows(seg, rows):
            r32 = lax.rem(seg, RES)
            for c in range(D // LANES):
                res_v[r32, pl.ds(c * LANES, LANES)] = zero16

            @plsc.parallel_loop(0, L, 8, unroll=2)
            def _(row):
                for c in range(D // 32):
                    ps = [rows[row + 2 * p, pl.ds(c * 32, 32)]
                          + rows[row + 2 * p + 1, pl.ds(c * 32, 32)]
                          for p in range(4)]
                    gs = [plsc.unpack(s, format=plsc.PackFormat.INTERLEAVED)
                          for s in ps]
                    s0 = (gs[0][0] + gs[1][0]) + (gs[2][0] + gs[3][0])
                    s1 = (gs[0][1] + gs[1][1]) + (gs[2][1] + gs[3][1])
                    plsc.addupdate(res_v.at[r32, pl.ds(c * 32, LANES)], s0)
                    plsc.addupdate(
                        res_v.at[r32, pl.ds(c * 32 + LANES, LANES)], s1)

        rows_l = (rows0, rows1, rows2, rows3, rows4)
        sems_l = (sem0, sem1, sem2, sem3, sem4)
        NB = len(rows_l)

        for b in range(NB - 1):
            g_start(b, rows_l[b], sems_l[b])

        def seg_body(seg, _):
            for q in range(NB):
                nb = (q + NB - 1) % NB

                @pl.when(lax.rem(seg, NB) == q)
                def _(q=q, nb=nb):
                    @pl.when(seg < seg_per_w - (NB - 1))
                    def _():
                        g_start(seg + NB - 1, rows_l[nb], sems_l[nb])
                    g_wait(rows_l[q], sems_l[q])
                    seg_sum_rows(seg, rows_l[q])

            @pl.when(lax.rem(seg, RES) == RES - 1)
            def _():
                blk = wseg + (seg // RES) * RES
                pltpu.sync_copy(res_v, out.at[pl.ds(blk, RES)])
            return 0

        lax.fori_loop(0, seg_per_w, seg_body, 0)

    return seg_sum


def _matmul_block(sa_ref, sr_ref, w_ref, out_ref):
    out_ref[:, :D] = jnp.dot(sa_ref[:], w_ref[:],
                             preferred_element_type=jnp.float32)
    out_ref[:, D:] = jnp.dot(sr_ref[:], w_ref[:],
                             preferred_element_type=jnp.float32)


@jax.jit
def kernel(added_sequences, removed_sequences, embed_table, W_prenoise):
    B, L = added_sequences.shape
    V, d = embed_table.shape
    idx = jnp.concatenate([added_sequences, removed_sequences], axis=0)
    idx = idx.astype(jnp.int32).reshape(2 * B * L)

    packed = _make_convert(V)(embed_table)
    sums = _make_seg_sum(2 * B, L, V)(packed, idx)

    bm = 512
    nblk = B // bm
    out = pl.pallas_call(
        _matmul_block,
        out_shape=jax.ShapeDtypeStruct((B, 2 * D), jnp.float32),
        grid=(nblk,),
        in_specs=[
            pl.BlockSpec((bm, D), lambda i: (i, 0)),
            pl.BlockSpec((bm, D), lambda i: (i + nblk, 0)),
            pl.BlockSpec((D, D), lambda i: (0, 0)),
        ],
        out_specs=pl.BlockSpec((bm, 2 * D), lambda i: (i, 0)),
    )(sums, sums, W_prenoise.T)
    return out


# speedup vs baseline: 1.1128x; 1.1128x over previous
import functools

import jax
import jax.numpy as jnp
from jax import lax
from jax.experimental import pallas as pl
from jax.experimental.pallas import tpu as pltpu
from jax.experimental.pallas import tpu_sc as plsc

NC, NS, LANES = 2, 16, 16
NW = NC * NS
D = 128
HA, HB = 96, 104
RES = 8
_SC_PARAMS = pltpu.CompilerParams(use_tc_tiling_on_sc=False,
                                  needs_layout_passes=False)


def _make_convert(V):
    rows_per_w = V // NW
    CH = 125
    nch = rows_per_w // CH
    assert rows_per_w % CH == 0
    mesh = plsc.VectorSubcoreMesh(core_axis_name="c", subcore_axis_name="s")

    @functools.partial(
        pl.kernel,
        out_type=jax.ShapeDtypeStruct((V, D), jnp.bfloat16),
        mesh=mesh,
        compiler_params=_SC_PARAMS,
        scratch_types=[
            pltpu.VMEM((3, CH, D), jnp.float32),
            pltpu.VMEM((3, CH, D), jnp.bfloat16),
            pltpu.SemaphoreType.DMA,
            pltpu.SemaphoreType.DMA,
            pltpu.SemaphoreType.DMA,
            pltpu.SemaphoreType.DMA,
            pltpu.SemaphoreType.DMA,
            pltpu.SemaphoreType.DMA,
        ],
    )
    def convert(table, out, in_v, out_v, si0, si1, si2, so0, so1, so2):
        wid = lax.axis_index("s") * NC + lax.axis_index("c")
        base = wid * rows_per_w
        sis = (si0, si1, si2)
        sos = (so0, so1, so2)

        def in_start(k, b):
            pltpu.make_async_copy(table.at[pl.ds(base + k * CH, CH)],
                                  in_v.at[b], sis[b]).start()

        def in_wait(b):
            pltpu.make_async_copy(table.at[pl.ds(base, CH)],
                                  in_v.at[b], sis[b]).wait()

        def out_start(k, b):
            pltpu.make_async_copy(out_v.at[b],
                                  out.at[pl.ds(base + k * CH, CH)],
                                  sos[b]).start()

        def out_wait(b):
            pltpu.make_async_copy(out_v.at[b],
                                  out.at[pl.ds(base, CH)], sos[b]).wait()

        def convert_chunk(b):
            @plsc.parallel_loop(0, CH, 1, unroll=5)
            def _(r):
                for c in range(D // 32):
                    g0 = in_v[b, r, pl.ds(c * 32, LANES)]
                    g1 = in_v[b, r, pl.ds(c * 32 + LANES, LANES)]
                    out_v[b, r, pl.ds(c * 32, 32)] = plsc.pack(
                        g0, g1, format=plsc.PackFormat.INTERLEAVED)

        in_start(0, 0)
        in_start(1, 1)

        def chunk_body(k, _):
            for q in range(3):
                @pl.when(lax.rem(k, 3) == q)
                def _(q=q):
                    @pl.when(k < nch - 2)
                    def _():
                        in_start(k + 2, (q + 2) % 3)
                    in_wait(q)
                    @pl.when(k >= 3)
                    def _():
                        out_wait(q)
                    convert_chunk(q)
                    out_start(k, q)
            return 0

        lax.fori_loop(0, nch, chunk_body, 0)
        for k in (nch - 3, nch - 2, nch - 1):
            out_wait(k % 3)

    return convert


def _make_seg_sum(S, L, V):
    assert L == HA + HB
    seg_per_w = S // NW
    npairs = seg_per_w // 2
    mesh = plsc.VectorSubcoreMesh(core_axis_name="c", subcore_axis_name="s")

    @functools.partial(
        pl.kernel,
        out_type=jax.ShapeDtypeStruct((S, D), jnp.float32),
        mesh=mesh,
        compiler_params=_SC_PARAMS,
        scratch_types=[
            pltpu.VMEM((seg_per_w * L,), jnp.int32),
            pltpu.VMEM((L, D), jnp.bfloat16),
            pltpu.VMEM((L, D), jnp.bfloat16),
            pltpu.VMEM((L, D), jnp.bfloat16),
            pltpu.VMEM((L, D), jnp.bfloat16),
            pltpu.VMEM((L, D), jnp.bfloat16),
            pltpu.VMEM((L, D), jnp.bfloat16),
            pltpu.VMEM((RES, D), jnp.float32),
            pltpu.SemaphoreType.DMA,
            pltpu.SemaphoreType.DMA,
            pltpu.SemaphoreType.DMA,
            pltpu.SemaphoreType.DMA,
            pltpu.SemaphoreType.DMA,
            pltpu.SemaphoreType.DMA,
        ],
    )
    def seg_sum(table, idx, out, idx_v, rows0, rows1, rows2, rows3, rows4,
                rows5, res_v, sem0, sem1, sem2, sem3, sem4, sem5):
        wid = lax.axis_index("s") * NC + lax.axis_index("c")
        wseg = wid * seg_per_w

        pltpu.sync_copy(idx.at[pl.ds(wseg * L, seg_per_w * L)], idx_v)

        def g_start(seg, rows, sem):
            off = seg * L
            pltpu.make_async_copy(
                table.at[idx_v.at[pl.ds(off, HA)]],
                rows.at[pl.ds(0, HA)], sem).start()
            pltpu.make_async_copy(
                table.at[idx_v.at[pl.ds(off + HA, HB)]],
                rows.at[pl.ds(HA, HB)], sem).start()

        def g_wait(rows, sem):
            pltpu.make_async_copy(
                table.at[idx_v.at[pl.ds(0, HA)]],
                rows.at[pl.ds(0, HA)], sem).wait()
            pltpu.make_async_copy(
                table.at[idx_v.at[pl.ds(0, HB)]],
                rows.at[pl.ds(HA, HB)], sem).wait()

        zero16 = jnp.zeros((LANES,), jnp.float32)

        def _tree_sum(vs):
            while len(vs) > 1:
                vs = [vs[i] + vs[i + 1] for i in range(0, len(vs) - 1, 2)] \
                    + ([vs[-1]] if len(vs) % 2 else [])
            return vs[0]

        def seg_sum_rows(seg, rows):
            zero = tuple(zero16 for _ in range(D // LANES))

            def body(i, acc):
                acc = list(acc)
                for p in range(10):
                    row = i * 20 + 2 * p
                    for c in range(D // 32):
                        s = (rows[row, pl.ds(c * 32, 32)]
                             + rows[row + 1, pl.ds(c * 32, 32)])
                        g0, g1 = plsc.unpack(
                            s, format=plsc.PackFormat.INTERLEAVED)
                        acc[2 * c] = acc[2 * c] + g0
                        acc[2 * c + 1] = acc[2 * c + 1] + g1
                return tuple(acc)

            accs = lax.fori_loop(0, L // 20, body, zero)
            r32 = lax.rem(seg, RES)
            for c in range(D // 32):
                res_v[r32, pl.ds(c * 32, LANES)] = accs[2 * c]
                res_v[r32, pl.ds(c * 32 + LANES, LANES)] = accs[2 * c + 1]

        rows_l = (rows0, rows1, rows2, rows3, rows4, rows5)
        sems_l = (sem0, sem1, sem2, sem3, sem4, sem5)
        NB = len(rows_l)

        for b in range(NB - 1):
            g_start(b, rows_l[b], sems_l[b])

        def seg_body(seg, _):
            for q in range(NB):
                nb = (q + NB - 1) % NB

                @pl.when(lax.rem(seg, NB) == q)
                def _(q=q, nb=nb):
                    @pl.when(seg < seg_per_w - (NB - 1))
                    def _():
                        g_start(seg + NB - 1, rows_l[nb], sems_l[nb])
                    g_wait(rows_l[q], sems_l[q])
                    seg_sum_rows(seg, rows_l[q])

            @pl.when(lax.rem(seg, RES) == RES - 1)
            def _():
                blk = wseg + (seg // RES) * RES
                pltpu.sync_copy(res_v, out.at[pl.ds(blk, RES)])
            return 0

        lax.fori_loop(0, seg_per_w, seg_body, 0)

    return seg_sum


def _matmul_block(sa_ref, sr_ref, w_ref, out_ref):
    out_ref[:, :D] = jnp.dot(sa_ref[:], w_ref[:],
                             preferred_element_type=jnp.float32)
    out_ref[:, D:] = jnp.dot(sr_ref[:], w_ref[:],
                             preferred_element_type=jnp.float32)


@jax.jit
def kernel(added_sequences, removed_sequences, embed_table, W_prenoise):
    B, L = added_sequences.shape
    V, d = embed_table.shape
    idx = jnp.concatenate([added_sequences, removed_sequences], axis=0)
    idx = idx.astype(jnp.int32).reshape(2 * B * L)

    packed = _make_convert(V)(embed_table)
    sums = _make_seg_sum(2 * B, L, V)(packed, idx)

    bm = 512
    nblk = B // bm
    out = pl.pallas_call(
        _matmul_block,
        out_shape=jax.ShapeDtypeStruct((B, 2 * D), jnp.float32),
        grid=(nblk,),
        in_specs=[
            pl.BlockSpec((bm, D), lambda i: (i, 0)),
            pl.BlockSpec((bm, D), lambda i: (i + nblk, 0)),
            pl.BlockSpec((D, D), lambda i: (0, 0)),
        ],
        out_specs=pl.BlockSpec((bm, 2 * D), lambda i: (i, 0)),
    )(sums, sums, W_prenoise.T)
    return out


# speedup vs baseline: 1.1884x; 1.0679x over previous
import functools

import jax
import jax.numpy as jnp
from jax import lax
from jax.experimental import pallas as pl
from jax.experimental.pallas import tpu as pltpu
from jax.experimental.pallas import tpu_sc as plsc

NC, NS, LANES = 2, 16, 16
NW = NC * NS
D = 128
HA, HB = 96, 104
RES = 16
_SC_PARAMS = pltpu.CompilerParams(use_tc_tiling_on_sc=False,
                                  needs_layout_passes=False)


def _make_convert(V):
    rows_per_w = V // NW
    CH = 125
    nch = rows_per_w // CH
    assert rows_per_w % CH == 0
    mesh = plsc.VectorSubcoreMesh(core_axis_name="c", subcore_axis_name="s")

    @functools.partial(
        pl.kernel,
        out_type=jax.ShapeDtypeStruct((V, D), jnp.bfloat16),
        mesh=mesh,
        compiler_params=_SC_PARAMS,
        scratch_types=[
            pltpu.VMEM((3, CH, D), jnp.float32),
            pltpu.VMEM((3, CH, D), jnp.bfloat16),
            pltpu.SemaphoreType.DMA,
            pltpu.SemaphoreType.DMA,
            pltpu.SemaphoreType.DMA,
            pltpu.SemaphoreType.DMA,
            pltpu.SemaphoreType.DMA,
            pltpu.SemaphoreType.DMA,
        ],
    )
    def convert(table, out, in_v, out_v, si0, si1, si2, so0, so1, so2):
        wid = lax.axis_index("s") * NC + lax.axis_index("c")
        base = wid * rows_per_w
        sis = (si0, si1, si2)
        sos = (so0, so1, so2)

        def in_start(k, b):
            pltpu.make_async_copy(table.at[pl.ds(base + k * CH, CH)],
                                  in_v.at[b], sis[b]).start()

        def in_wait(b):
            pltpu.make_async_copy(table.at[pl.ds(base, CH)],
                                  in_v.at[b], sis[b]).wait()

        def out_start(k, b):
            pltpu.make_async_copy(out_v.at[b],
                                  out.at[pl.ds(base + k * CH, CH)],
                                  sos[b]).start()

        def out_wait(b):
            pltpu.make_async_copy(out_v.at[b],
                                  out.at[pl.ds(base, CH)], sos[b]).wait()

        def convert_chunk(b):
            @plsc.parallel_loop(0, CH, 1, unroll=5)
            def _(r):
                for c in range(D // 32):
                    g0 = in_v[b, r, pl.ds(c * 32, LANES)]
                    g1 = in_v[b, r, pl.ds(c * 32 + LANES, LANES)]
                    out_v[b, r, pl.ds(c * 32, 32)] = plsc.pack(
                        g0, g1, format=plsc.PackFormat.INTERLEAVED)

        in_start(0, 0)
        in_start(1, 1)

        def chunk_body(k, _):
            for q in range(3):
                @pl.when(lax.rem(k, 3) == q)
                def _(q=q):
                    @pl.when(k < nch - 2)
                    def _():
                        in_start(k + 2, (q + 2) % 3)
                    in_wait(q)
                    @pl.when(k >= 3)
                    def _():
                        out_wait(q)
                    convert_chunk(q)
                    out_start(k, q)
            return 0

        lax.fori_loop(0, nch, chunk_body, 0)
        for k in (nch - 3, nch - 2, nch - 1):
            out_wait(k % 3)

    return convert


def _make_seg_sum(S, L, V):
    assert L == HA + HB
    seg_per_w = S // NW
    npairs = seg_per_w // 2
    mesh = plsc.VectorSubcoreMesh(core_axis_name="c", subcore_axis_name="s")

    @functools.partial(
        pl.kernel,
        out_type=jax.ShapeDtypeStruct((S, D), jnp.float32),
        mesh=mesh,
        compiler_params=_SC_PARAMS,
        scratch_types=[
            pltpu.VMEM((seg_per_w * L,), jnp.int32),
            pltpu.VMEM((L, D), jnp.bfloat16),
            pltpu.VMEM((L, D), jnp.bfloat16),
            pltpu.VMEM((L, D), jnp.bfloat16),
            pltpu.VMEM((L, D), jnp.bfloat16),
            pltpu.VMEM((L, D), jnp.bfloat16),
            pltpu.VMEM((RES, D), jnp.float32),
            pltpu.SemaphoreType.DMA,
            pltpu.SemaphoreType.DMA,
            pltpu.SemaphoreType.DMA,
            pltpu.SemaphoreType.DMA,
            pltpu.SemaphoreType.DMA,
        ],
    )
    def seg_sum(table, idx_a, idx_r, out, idx_v, rows0, rows1, rows2, rows3,
                rows4, res_v, sem0, sem1, sem2, sem3, sem4):
        wid = lax.axis_index("s") * NC + lax.axis_index("c")
        half = seg_per_w // 2

        pltpu.sync_copy(idx_a.at[pl.ds(wid * half * L, half * L)],
                        idx_v.at[pl.ds(0, half * L)])
        pltpu.sync_copy(idx_r.at[pl.ds(wid * half * L, half * L)],
                        idx_v.at[pl.ds(half * L, half * L)])

        def g_start(seg, rows, sem):
            off = seg * L
            pltpu.make_async_copy(
                table.at[idx_v.at[pl.ds(off, HA)]],
                rows.at[pl.ds(0, HA)], sem).start()
            pltpu.make_async_copy(
                table.at[idx_v.at[pl.ds(off + HA, HB)]],
                rows.at[pl.ds(HA, HB)], sem).start()

        def g_wait(rows, sem):
            pltpu.make_async_copy(
                table.at[idx_v.at[pl.ds(0, HA)]],
                rows.at[pl.ds(0, HA)], sem).wait()
            pltpu.make_async_copy(
                table.at[idx_v.at[pl.ds(0, HB)]],
                rows.at[pl.ds(HA, HB)], sem).wait()

        zero16 = jnp.zeros((LANES,), jnp.float32)

        def _tree_sum(vs):
            while len(vs) > 1:
                vs = [vs[i] + vs[i + 1] for i in range(0, len(vs) - 1, 2)] \
                    + ([vs[-1]] if len(vs) % 2 else [])
            return vs[0]

        def seg_sum_rows(seg, rows):
            zero = tuple(zero16 for _ in range(D // LANES))

            def body(i, acc):
                acc = list(acc)
                for p in range(10):
                    row = i * 20 + 2 * p
                    for c in range(D // 32):
                        s = (rows[row, pl.ds(c * 32, 32)]
                             + rows[row + 1, pl.ds(c * 32, 32)])
                        g0, g1 = plsc.unpack(
                            s, format=plsc.PackFormat.INTERLEAVED)
                        acc[2 * c] = acc[2 * c] + g0
                        acc[2 * c + 1] = acc[2 * c + 1] + g1
                return tuple(acc)

            accs = lax.fori_loop(0, L // 20, body, zero)
            r32 = lax.rem(seg, RES)
            for c in range(D // 32):
                res_v[r32, pl.ds(c * 32, LANES)] = accs[2 * c]
                res_v[r32, pl.ds(c * 32 + LANES, LANES)] = accs[2 * c + 1]

        rows_l = (rows0, rows1, rows2, rows3, rows4)
        sems_l = (sem0, sem1, sem2, sem3, sem4)
        NB = len(rows_l)

        for b in range(NB - 1):
            g_start(b, rows_l[b], sems_l[b])

        def seg_body(seg, _):
            for q in range(NB):
                nb = (q + NB - 1) % NB

                @pl.when(lax.rem(seg, NB) == q)
                def _(q=q, nb=nb):
                    @pl.when(seg < seg_per_w - (NB - 1))
                    def _():
                        g_start(seg + NB - 1, rows_l[nb], sems_l[nb])
                    g_wait(rows_l[q], sems_l[q])
                    seg_sum_rows(seg, rows_l[q])

            @pl.when(lax.rem(seg, RES) == RES - 1)
            def _():
                start = (seg // RES) * RES
                blk = jnp.where(start < half, wid * half + start,
                                S // 2 + wid * half + (start - half))
                pltpu.sync_copy(res_v, out.at[pl.ds(blk, RES)])
            return 0

        lax.fori_loop(0, seg_per_w, seg_body, 0)

    return seg_sum


def _matmul_block(sa_ref, sr_ref, w_ref, out_ref):
    out_ref[:, :D] = jnp.dot(sa_ref[:], w_ref[:],
                             preferred_element_type=jnp.float32)
    out_ref[:, D:] = jnp.dot(sr_ref[:], w_ref[:],
                             preferred_element_type=jnp.float32)


@jax.jit
def kernel(added_sequences, removed_sequences, embed_table, W_prenoise):
    B, L = added_sequences.shape
    V, d = embed_table.shape
    idx_a = added_sequences.astype(jnp.int32).reshape(B * L)
    idx_r = removed_sequences.astype(jnp.int32).reshape(B * L)

    packed = _make_convert(V)(embed_table)
    sums = _make_seg_sum(2 * B, L, V)(packed, idx_a, idx_r)

    bm = 512
    nblk = B // bm
    out = pl.pallas_call(
        _matmul_block,
        out_shape=jax.ShapeDtypeStruct((B, 2 * D), jnp.float32),
        grid=(nblk,),
        in_specs=[
            pl.BlockSpec((bm, D), lambda i: (i, 0)),
            pl.BlockSpec((bm, D), lambda i: (i + nblk, 0)),
            pl.BlockSpec((D, D), lambda i: (0, 0)),
        ],
        out_specs=pl.BlockSpec((bm, 2 * D), lambda i: (i, 0)),
    )(sums, sums, W_prenoise.T)
    return out


# speedup vs baseline: 1.1905x; 1.0018x over previous
import functools

import jax
import jax.numpy as jnp
from jax import lax
from jax.experimental import pallas as pl
from jax.experimental.pallas import tpu as pltpu
from jax.experimental.pallas import tpu_sc as plsc

NC, NS, LANES = 2, 16, 16
NW = NC * NS
D = 128
HA, HB = 96, 104
RES = 16
_SC_PARAMS = pltpu.CompilerParams(use_tc_tiling_on_sc=False,
                                  needs_layout_passes=False)


def _make_convert(V):
    rows_per_w = V // NW
    CH = 125
    nch = rows_per_w // CH
    assert rows_per_w % CH == 0
    mesh = plsc.VectorSubcoreMesh(core_axis_name="c", subcore_axis_name="s")

    @functools.partial(
        pl.kernel,
        out_type=jax.ShapeDtypeStruct((V, D), jnp.bfloat16),
        mesh=mesh,
        compiler_params=_SC_PARAMS,
        scratch_types=[
            pltpu.VMEM((3, CH, D), jnp.float32),
            pltpu.VMEM((3, CH, D), jnp.bfloat16),
            pltpu.SemaphoreType.DMA,
            pltpu.SemaphoreType.DMA,
            pltpu.SemaphoreType.DMA,
            pltpu.SemaphoreType.DMA,
            pltpu.SemaphoreType.DMA,
            pltpu.SemaphoreType.DMA,
        ],
    )
    def convert(table, out, in_v, out_v, si0, si1, si2, so0, so1, so2):
        wid = lax.axis_index("s") * NC + lax.axis_index("c")
        base = wid * rows_per_w
        sis = (si0, si1, si2)
        sos = (so0, so1, so2)

        def in_start(k, b):
            pltpu.make_async_copy(table.at[pl.ds(base + k * CH, CH)],
                                  in_v.at[b], sis[b]).start()

        def in_wait(b):
            pltpu.make_async_copy(table.at[pl.ds(base, CH)],
                                  in_v.at[b], sis[b]).wait()

        def out_start(k, b):
            pltpu.make_async_copy(out_v.at[b],
                                  out.at[pl.ds(base + k * CH, CH)],
                                  sos[b]).start()

        def out_wait(b):
            pltpu.make_async_copy(out_v.at[b],
                                  out.at[pl.ds(base, CH)], sos[b]).wait()

        def convert_chunk(b):
            @plsc.parallel_loop(0, CH, 1, unroll=5)
            def _(r):
                for c in range(D // 32):
                    g0 = in_v[b, r, pl.ds(c * 32, LANES)]
                    g1 = in_v[b, r, pl.ds(c * 32 + LANES, LANES)]
                    out_v[b, r, pl.ds(c * 32, 32)] = plsc.pack(
                        g0, g1, format=plsc.PackFormat.INTERLEAVED)

        in_start(0, 0)
        in_start(1, 1)

        def chunk_body(k, _):
            for q in range(3):
                @pl.when(lax.rem(k, 3) == q)
                def _(q=q):
                    @pl.when(k < nch - 2)
                    def _():
                        in_start(k + 2, (q + 2) % 3)
                    in_wait(q)
                    @pl.when(k >= 3)
                    def _():
                        out_wait(q)
                    convert_chunk(q)
                    out_start(k, q)
            return 0

        lax.fori_loop(0, nch, chunk_body, 0)
        for k in (nch - 3, nch - 2, nch - 1):
            out_wait(k % 3)

    return convert


def _make_seg_sum(S, L, V):
    assert L == HA + HB
    seg_per_w = S // NW
    mesh = plsc.VectorSubcoreMesh(core_axis_name="c", subcore_axis_name="s")

    @functools.partial(
        pl.kernel,
        out_type=jax.ShapeDtypeStruct((S, D), jnp.float32),
        mesh=mesh,
        compiler_params=_SC_PARAMS,
        scratch_types=[
            pltpu.VMEM((seg_per_w * L,), jnp.int32),
            pltpu.VMEM((L, D), jnp.bfloat16),
            pltpu.VMEM((L, D), jnp.bfloat16),
            pltpu.VMEM((L, D), jnp.bfloat16),
            pltpu.VMEM((L, D), jnp.bfloat16),
            pltpu.VMEM((L, D), jnp.bfloat16),
            pltpu.VMEM((RES, D), jnp.float32),
            pltpu.SemaphoreType.DMA,
            pltpu.SemaphoreType.DMA,
            pltpu.SemaphoreType.DMA,
            pltpu.SemaphoreType.DMA,
            pltpu.SemaphoreType.DMA,
        ],
    )
    def seg_sum(table, idx_a, idx_r, out, idx_v, rows0, rows1, rows2, rows3,
                rows4, res_v, sem0, sem1, sem2, sem3, sem4):
        wid = lax.axis_index("s") * NC + lax.axis_index("c")
        half = seg_per_w // 2

        pltpu.sync_copy(idx_a.at[pl.ds(wid * half * L, half * L)],
                        idx_v.at[pl.ds(0, half * L)])
        pltpu.sync_copy(idx_r.at[pl.ds(wid * half * L, half * L)],
                        idx_v.at[pl.ds(half * L, half * L)])

        def g_start(seg, rows, sem):
            off = seg * L
            pltpu.make_async_copy(
                table.at[idx_v.at[pl.ds(off, HA)]],
                rows.at[pl.ds(0, HA)], sem).start()
            pltpu.make_async_copy(
                table.at[idx_v.at[pl.ds(off + HA, HB)]],
                rows.at[pl.ds(HA, HB)], sem).start()

        def g_wait(rows, sem):
            pltpu.make_async_copy(
                table.at[idx_v.at[pl.ds(0, HA)]],
                rows.at[pl.ds(0, HA)], sem).wait()
            pltpu.make_async_copy(
                table.at[idx_v.at[pl.ds(0, HB)]],
                rows.at[pl.ds(HA, HB)], sem).wait()

        zero16 = jnp.zeros((LANES,), jnp.float32)

        def _tree_sum(vs):
            while len(vs) > 1:
                vs = [vs[i] + vs[i + 1] for i in range(0, len(vs) - 1, 2)] \
                    + ([vs[-1]] if len(vs) % 2 else [])
            return vs[0]

        def seg_sum_rows(seg, rows):
            zero = tuple(zero16 for _ in range(D // LANES))

            def body(i, acc):
                acc = list(acc)
                for p in range(10):
                    row = i * 20 + 2 * p
                    for c in range(D // 32):
                        s = (rows[row, pl.ds(c * 32, 32)]
                             + rows[row + 1, pl.ds(c * 32, 32)])
                        g0, g1 = plsc.unpack(
                            s, format=plsc.PackFormat.INTERLEAVED)
                        acc[2 * c] = acc[2 * c] + g0
                        acc[2 * c + 1] = acc[2 * c + 1] + g1
                return tuple(acc)

            accs = lax.fori_loop(0, L // 20, body, zero)
            r32 = lax.rem(seg, RES)
            for c in range(D // 32):
                res_v[r32, pl.ds(c * 32, LANES)] = accs[2 * c]
                res_v[r32, pl.ds(c * 32 + LANES, LANES)] = accs[2 * c + 1]

        rows_l = (rows0, rows1, rows2, rows3, rows4)
        sems_l = (sem0, sem1, sem2, sem3, sem4)
        NB = len(rows_l)

        for b in range(NB - 1):
            g_start(b, rows_l[b], sems_l[b])

        def seg_body(seg, _):
            for q in range(NB):
                nb = (q + NB - 1) % NB

                @pl.when(lax.rem(seg, NB) == q)
                def _(q=q, nb=nb):
                    @pl.when(seg < seg_per_w - (NB - 1))
                    def _():
                        g_start(seg + NB - 1, rows_l[nb], sems_l[nb])
                    g_wait(rows_l[q], sems_l[q])
                    seg_sum_rows(seg, rows_l[q])

            @pl.when(lax.rem(seg, RES) == RES - 1)
            def _():
                start = (seg // RES) * RES
                blk = jnp.where(start < half, wid * half + start,
                                S // 2 + wid * half + (start - half))
                pltpu.sync_copy(res_v, out.at[pl.ds(blk, RES)])
            return 0

        lax.fori_loop(0, seg_per_w, seg_body, 0)

    return seg_sum


def _matmul_block(sa_ref, sr_ref, w_ref, out_ref):
    out_ref[:, :D] = jnp.dot(sa_ref[:], w_ref[:],
                             preferred_element_type=jnp.float32)
    out_ref[:, D:] = jnp.dot(sr_ref[:], w_ref[:],
                             preferred_element_type=jnp.float32)


@jax.jit
def kernel(added_sequences, removed_sequences, embed_table, W_prenoise):
    B, L = added_sequences.shape
    V, d = embed_table.shape
    idx_a = added_sequences.astype(jnp.int32).reshape(B * L)
    idx_r = removed_sequences.astype(jnp.int32).reshape(B * L)

    packed = _make_convert(V)(embed_table)
    sums = _make_seg_sum(2 * B, L, V)(packed, idx_a, idx_r)

    bm = 512
    nblk = B // bm
    out = pl.pallas_call(
        _matmul_block,
        out_shape=jax.ShapeDtypeStruct((B, 2 * D), jnp.float32),
        grid=(nblk,),
        in_specs=[
            pl.BlockSpec((bm, D), lambda i: (i, 0)),
            pl.BlockSpec((bm, D), lambda i: (i + nblk, 0)),
            pl.BlockSpec((D, D), lambda i: (0, 0)),
        ],
        out_specs=pl.BlockSpec((bm, 2 * D), lambda i: (i, 0)),
    )(sums, sums, W_prenoise.T)
    return out
